# Initial kernel scaffold; baseline (speedup 1.0000x reference)
import jax
import jax.numpy as jnp
from jax.experimental import pallas as pl


def kernel(x, edge_index, batch, W1a, b1a, W1b, b1b, W2a, b2a, W2b, b2b, W3a, b3a, W3b, b3b, Wc1, bc1, Wc2, bc2, Wc3, bc3):
    raise NotImplementedError("write your pallas kernel here")


# speedup vs baseline: 1.0945x; 1.0945x over previous
import functools

import jax
import jax.numpy as jnp
from jax.experimental import pallas as pl

N_NODES = 10000
N_GRAPHS = 64
N_EDGES = 320000
F = 128


def _proj_kernel(x_ref, w_ref, ba_ref, a_ref, b_ref):
    xb = x_ref[...]
    ab = jnp.dot(xb, w_ref[...], preferred_element_type=jnp.float32)
    a_ref[...] = ab[:, :F] + ba_ref[...]
    b_ref[...] = ab[:, F:]


def _node_proj(x, Wa, ba):
    U = Wa[:F] - Wa[F:]
    V = Wa[F:]
    W2 = jnp.concatenate([U, V], axis=1)
    blk = 2000
    grid = (N_NODES // blk,)
    return pl.pallas_call(
        _proj_kernel,
        grid=grid,
        in_specs=[
            pl.BlockSpec((blk, F), lambda i: (i, 0)),
            pl.BlockSpec((F, 2 * F), lambda i: (0, 0)),
            pl.BlockSpec((F,), lambda i: (0,)),
        ],
        out_specs=[
            pl.BlockSpec((blk, F), lambda i: (i, 0)),
            pl.BlockSpec((blk, F), lambda i: (i, 0)),
        ],
        out_shape=[
            jax.ShapeDtypeStruct((N_NODES, F), jnp.float32),
            jax.ShapeDtypeStruct((N_NODES, F), jnp.float32),
        ],
    )(x, W2, ba)


def _edge_mlp_kernel(z_ref, w_ref, b_ref, y_ref):
    h = jnp.maximum(z_ref[...], 0.0)
    y_ref[...] = jnp.dot(h, w_ref[...], preferred_element_type=jnp.float32) + b_ref[...]


def _edge_mlp(z, Wb, bb):
    blk = 2000
    grid = (N_EDGES // blk,)
    return pl.pallas_call(
        _edge_mlp_kernel,
        grid=grid,
        in_specs=[
            pl.BlockSpec((blk, F), lambda i: (i, 0)),
            pl.BlockSpec((F, F), lambda i: (0, 0)),
            pl.BlockSpec((F,), lambda i: (0,)),
        ],
        out_specs=pl.BlockSpec((blk, F), lambda i: (i, 0)),
        out_shape=jax.ShapeDtypeStruct((N_EDGES, F), jnp.float32),
    )(z, Wb, bb)


def kernel(x, edge_index, batch, W1a, b1a, W1b, b1b, W2a, b2a, W2b, b2b,
           W3a, b3a, W3b, b3b, Wc1, bc1, Wc2, bc2, Wc3, bc3):
    src = edge_index[0]
    dst = edge_index[1]

    def layer(xin, Wa, ba, Wb, bb):
        A, B = _node_proj(xin, Wa, ba)
        z = jnp.take(A, dst, axis=0) + jnp.take(B, src, axis=0)
        y = _edge_mlp(z, Wb, bb)
        out = jax.ops.segment_max(y, dst, num_segments=N_NODES)
        return jnp.maximum(out, 0.0)

    h1 = layer(x, W1a, b1a, W1b, b1b)
    h2 = layer(h1, W2a, b2a, W2b, b2b)
    h3 = layer(h2, W3a, b3a, W3b, b3b)

    ones = jnp.ones((N_NODES,), dtype=jnp.float32)
    counts = jax.ops.segment_sum(ones, batch, num_segments=N_GRAPHS)
    x_sum = jax.ops.segment_sum(h3, batch, num_segments=N_GRAPHS)
    x_mean = x_sum / jnp.clip(counts, 1.0)[:, None]
    x_max = jax.ops.segment_max(h3, batch, num_segments=N_GRAPHS)
    x_max = jnp.where(jnp.isfinite(x_max), x_max, 0.0)
    g = jnp.concatenate([x_mean, x_max], axis=1)
    h = jax.nn.relu(g @ Wc1 + bc1)
    h = jax.nn.relu(h @ Wc2 + bc2)
    out = h @ Wc3 + bc3
    return jnp.squeeze(out, axis=-1)


# speedup vs baseline: 1.8724x; 1.7107x over previous
import functools

import jax
import jax.numpy as jnp
from jax import lax
from jax.experimental import pallas as pl
from jax.experimental.pallas import tpu as pltpu
from jax.experimental.pallas import tpu_sc as plsc

N_NODES = 10000
N_GRAPHS = 64
N_EDGES = 320000
F = 128

NC, NS = 2, 16
NW = NC * NS
NPW = 320
NPAD = NW * NPW
GC = 128
NCHUNKS = N_EDGES // GC
SC_CH = 256

_mesh = plsc.VectorSubcoreMesh(core_axis_name="c", subcore_axis_name="s")


def _wid():
    return lax.axis_index("s") * NC + lax.axis_index("c")


def _proj_body(x_ref, w_ref, ba_ref, a_ref, b_ref):
    ab = jnp.dot(x_ref[...], w_ref[...], preferred_element_type=jnp.float32)
    a_ref[...] = ab[:, :F] + ba_ref[...]
    b_ref[...] = ab[:, F:]


def _node_proj(x, Wa, ba):
    U = Wa[:F] - Wa[F:]
    V = Wa[F:]
    W2 = jnp.concatenate([U, V], axis=1)
    blk = 2000
    return pl.pallas_call(
        _proj_body,
        grid=(N_NODES // blk,),
        in_specs=[
            pl.BlockSpec((blk, F), lambda i: (i, 0)),
            pl.BlockSpec((F, 2 * F), lambda i: (0, 0)),
            pl.BlockSpec((F,), lambda i: (0,)),
        ],
        out_specs=[
            pl.BlockSpec((blk, F), lambda i: (i, 0)),
            pl.BlockSpec((blk, F), lambda i: (i, 0)),
        ],
        out_shape=[
            jax.ShapeDtypeStruct((N_NODES, F), jnp.float32),
            jax.ShapeDtypeStruct((N_NODES, F), jnp.float32),
        ],
    )(x, W2, ba)


def _edge_mlp_body(z_ref, w_ref, b_ref, y_ref):
    h = jnp.maximum(z_ref[...], 0.0)
    y_ref[...] = jnp.dot(h, w_ref[...], preferred_element_type=jnp.float32) + b_ref[...]


def _edge_mlp(z, Wb, bb):
    blk = 2000
    return pl.pallas_call(
        _edge_mlp_body,
        grid=(N_EDGES // blk,),
        in_specs=[
            pl.BlockSpec((blk, F), lambda i: (i, 0)),
            pl.BlockSpec((F, F), lambda i: (0, 0)),
            pl.BlockSpec((F,), lambda i: (0,)),
        ],
        out_specs=pl.BlockSpec((blk, F), lambda i: (i, 0)),
        out_shape=jax.ShapeDtypeStruct((N_EDGES, F), jnp.float32),
    )(z, Wb, bb)


@functools.partial(
    pl.kernel,
    out_type=jax.ShapeDtypeStruct((N_EDGES, F), jnp.float32),
    mesh=_mesh,
    scratch_types=[
        pltpu.VMEM((GC,), jnp.int32),
        pltpu.VMEM((GC,), jnp.int32),
        pltpu.VMEM((GC, F), jnp.float32),
        pltpu.VMEM((GC, F), jnp.float32),
        pltpu.SemaphoreType.DMA,
        pltpu.SemaphoreType.DMA,
    ],
)
def _sc_gather_combine(a_hbm, b_hbm, src_hbm, dst_hbm, h_hbm,
                       sidx, didx, abuf, bbuf, sem_a, sem_b):
    w = _wid()
    nch = (NCHUNKS - 1 - w) // NW + 1

    def chunk(k, _):
        base = (w + k * NW) * GC
        pltpu.sync_copy(dst_hbm.at[pl.ds(base, GC)], didx)
        pltpu.sync_copy(src_hbm.at[pl.ds(base, GC)], sidx)
        cpa = pltpu.async_copy(a_hbm.at[didx], abuf, sem_a)
        cpb = pltpu.async_copy(b_hbm.at[sidx], bbuf, sem_b)
        cpa.wait()
        cpb.wait()

        def row(r, _):
            for j in range(8):
                sl = pl.ds(j * 16, 16)
                abuf[r, sl] = abuf[r, sl] + bbuf[r, sl]
            return 0

        lax.fori_loop(0, GC, row, 0)
        pltpu.sync_copy(abuf, h_hbm.at[pl.ds(base, GC)])
        return 0

    lax.fori_loop(0, nch, chunk, 0)


@functools.partial(
    pl.kernel,
    out_type=jax.ShapeDtypeStruct((NPAD, F), jnp.float32),
    mesh=_mesh,
    scratch_types=[
        pltpu.VMEM((40,), jnp.int32),
        pltpu.VMEM((SC_CH,), jnp.int32),
        pltpu.VMEM((SC_CH, F), jnp.float32),
        pltpu.VMEM((NPW + 1, F), jnp.float32),
    ],
)
def _sc_segmax(y_hbm, dst_hbm, bounds_hbm, x_hbm, bnd, dbuf, ybuf, acc):
    w = _wid()
    pltpu.sync_copy(bounds_hbm, bnd)
    nbase = w * NPW
    zero = jnp.zeros((16,), jnp.float32)

    def zrow(r, _):
        for j in range(8):
            acc[r, pl.ds(j * 16, 16)] = zero
        return 0

    lax.fori_loop(0, NPW + 1, zrow, 0)

    bv = bnd[pl.ds(w, 16)]
    lo = bv[0]
    hi = bv[1]
    lo8 = (lo // 8) * 8
    nch = (hi - lo8 + SC_CH - 1) // SC_CH

    def chunk(i, _):
        base = jnp.minimum(lo8 + i * SC_CH, N_EDGES - SC_CH)
        pltpu.sync_copy(dst_hbm.at[pl.ds(base, SC_CH)], dbuf)
        pltpu.sync_copy(y_hbm.at[pl.ds(base, SC_CH)], ybuf)

        def grp(g, _):
            dvec = dbuf[pl.ds(g * 16, 16)]
            for ii in range(16):
                d = dvec[ii]
                r = d - nbase
                r = jnp.where((r >= 0) & (r < NPW), r, NPW)
                e = g * 16 + ii
                for j in range(8):
                    sl = pl.ds(j * 16, 16)
                    acc[r, sl] = jnp.maximum(acc[r, sl], ybuf[e, sl])
            return 0

        lax.fori_loop(0, SC_CH // 16, grp, 0)
        return 0

    lax.fori_loop(0, nch, chunk, 0)
    pltpu.sync_copy(acc.at[pl.ds(0, NPW)], x_hbm.at[pl.ds(nbase, NPW)])


def kernel(x, edge_index, batch, W1a, b1a, W1b, b1b, W2a, b2a, W2b, b2b,
           W3a, b3a, W3b, b3b, Wc1, bc1, Wc2, bc2, Wc3, bc3):
    src = edge_index[0]
    dst = edge_index[1]
    sdst, ssrc = lax.sort((dst, src), num_keys=1)
    starts = jnp.arange(33, dtype=jnp.int32) * NPW
    bounds = jnp.searchsorted(sdst, starts).astype(jnp.int32)
    bounds = jnp.zeros((40,), jnp.int32).at[:33].set(bounds)

    def layer(xin, Wa, ba, Wb, bb):
        A, B = _node_proj(xin, Wa, ba)
        H = _sc_gather_combine(A, B, ssrc, sdst)
        Y = _edge_mlp(H, Wb, bb)
        xp = _sc_segmax(Y, sdst, bounds)
        return xp[:N_NODES]

    h1 = layer(x, W1a, b1a, W1b, b1b)
    h2 = layer(h1, W2a, b2a, W2b, b2b)
    h3 = layer(h2, W3a, b3a, W3b, b3b)

    ones = jnp.ones((N_NODES,), dtype=jnp.float32)
    counts = jax.ops.segment_sum(ones, batch, num_segments=N_GRAPHS)
    x_sum = jax.ops.segment_sum(h3, batch, num_segments=N_GRAPHS)
    x_mean = x_sum / jnp.clip(counts, 1.0)[:, None]
    x_max = jax.ops.segment_max(h3, batch, num_segments=N_GRAPHS)
    x_max = jnp.where(jnp.isfinite(x_max), x_max, 0.0)
    g = jnp.concatenate([x_mean, x_max], axis=1)
    h = jax.nn.relu(g @ Wc1 + bc1)
    h = jax.nn.relu(h @ Wc2 + bc2)
    out = h @ Wc3 + bc3
    return jnp.squeeze(out, axis=-1)


# speedup vs baseline: 2.9513x; 1.5762x over previous
import functools

import jax
import jax.numpy as jnp
from jax import lax
from jax.experimental import pallas as pl
from jax.experimental.pallas import tpu as pltpu
from jax.experimental.pallas import tpu_sc as plsc

N_NODES = 10000
N_GRAPHS = 64
N_EDGES = 320000
F = 128

NC, NS = 2, 16
NW = NC * NS
NPW = 320
NPAD = NW * NPW
GC = 128
NCHUNKS = N_EDGES // GC
SC_CH = 256

_mesh = plsc.VectorSubcoreMesh(core_axis_name="c", subcore_axis_name="s")


def _wid():
    return lax.axis_index("s") * NC + lax.axis_index("c")


def _proj_body(x_ref, w_ref, ba_ref, a_ref, b_ref):
    ab = jnp.dot(x_ref[...], w_ref[...], preferred_element_type=jnp.float32)
    a_ref[...] = ab[:, :F] + ba_ref[...]
    b_ref[...] = ab[:, F:]


def _node_proj(x, Wa, ba):
    U = Wa[:F] - Wa[F:]
    V = Wa[F:]
    W2 = jnp.concatenate([U, V], axis=1)
    blk = 2000
    return pl.pallas_call(
        _proj_body,
        grid=(N_NODES // blk,),
        in_specs=[
            pl.BlockSpec((blk, F), lambda i: (i, 0)),
            pl.BlockSpec((F, 2 * F), lambda i: (0, 0)),
            pl.BlockSpec((F,), lambda i: (0,)),
        ],
        out_specs=[
            pl.BlockSpec((blk, F), lambda i: (i, 0)),
            pl.BlockSpec((blk, F), lambda i: (i, 0)),
        ],
        out_shape=[
            jax.ShapeDtypeStruct((N_NODES, F), jnp.float32),
            jax.ShapeDtypeStruct((N_NODES, F), jnp.float32),
        ],
    )(x, W2, ba)


def _edge_mlp_body(z_ref, w_ref, b_ref, y_ref):
    h = jnp.maximum(z_ref[...], 0.0)
    y_ref[...] = jnp.dot(h, w_ref[...], preferred_element_type=jnp.float32) + b_ref[...]


def _edge_mlp(z, Wb, bb):
    blk = 2000
    return pl.pallas_call(
        _edge_mlp_body,
        grid=(N_EDGES // blk,),
        in_specs=[
            pl.BlockSpec((blk, F), lambda i: (i, 0)),
            pl.BlockSpec((F, F), lambda i: (0, 0)),
            pl.BlockSpec((F,), lambda i: (0,)),
        ],
        out_specs=pl.BlockSpec((blk, F), lambda i: (i, 0)),
        out_shape=jax.ShapeDtypeStruct((N_EDGES, F), jnp.float32),
    )(z, Wb, bb)


@functools.partial(
    pl.kernel,
    out_type=jax.ShapeDtypeStruct((N_EDGES, F), jnp.float32),
    mesh=_mesh,
    scratch_types=[
        pltpu.VMEM((80 * GC,), jnp.int32),
        pltpu.VMEM((80 * GC,), jnp.int32),
        pltpu.VMEM((GC, F), jnp.float32),
        pltpu.VMEM((GC, F), jnp.float32),
        pltpu.VMEM((GC, F), jnp.float32),
        pltpu.VMEM((GC, F), jnp.float32),
        pltpu.SemaphoreType.DMA,
        pltpu.SemaphoreType.DMA,
        pltpu.SemaphoreType.DMA,
        pltpu.SemaphoreType.DMA,
    ],
)
def _sc_gather_combine(a_hbm, b_hbm, src_hbm, dst_hbm, h_hbm,
                       sidx2, didx2, abuf0, abuf1, bbuf0, bbuf1,
                       sa0, sa1, sb0, sb1):
    w = _wid()
    clo = 2 * ((w * (NCHUNKS // 2)) // NW)
    chi = 2 * (((w + 1) * (NCHUNKS // 2)) // NW)
    nch = chi - clo

    pltpu.sync_copy(dst_hbm.at[pl.ds(clo * GC, 80 * GC)], didx2)
    pltpu.sync_copy(src_hbm.at[pl.ds(clo * GC, 80 * GC)], sidx2)

    abufs = (abuf0, abuf1)
    bbufs = (bbuf0, bbuf1)
    sas = (sa0, sa1)
    sbs = (sb0, sb1)

    def issue(kk, p):
        pltpu.async_copy(a_hbm.at[didx2.at[pl.ds(kk * GC, GC)]], abufs[p], sas[p])
        pltpu.async_copy(b_hbm.at[sidx2.at[pl.ds(kk * GC, GC)]], bbufs[p], sbs[p])

    issue(0, 0)
    issue(1, 1)

    def pair(i, _):
        for p in range(2):
            kk = 2 * i + p
            pltpu.make_async_copy(a_hbm.at[pl.ds(0, GC)], abufs[p], sas[p]).wait()
            pltpu.make_async_copy(b_hbm.at[pl.ds(0, GC)], bbufs[p], sbs[p]).wait()

            def row(r, _, p=p):
                for j in range(8):
                    sl = pl.ds(j * 16, 16)
                    abufs[p][r, sl] = abufs[p][r, sl] + bbufs[p][r, sl]
                return 0

            lax.fori_loop(0, GC, row, 0)
            pltpu.sync_copy(abufs[p], h_hbm.at[pl.ds((clo + kk) * GC, GC)])

            @pl.when(kk + 2 < nch)
            def _(kk=kk, p=p):
                issue(kk + 2, p)
        return 0

    lax.fori_loop(0, nch // 2, pair, 0)


@functools.partial(
    pl.kernel,
    out_type=jax.ShapeDtypeStruct((NPAD, F), jnp.float32),
    mesh=_mesh,
    scratch_types=[
        pltpu.VMEM((40,), jnp.int32),
        pltpu.VMEM((SC_CH,), jnp.int32),
        pltpu.VMEM((SC_CH,), jnp.int32),
        pltpu.VMEM((SC_CH, F), jnp.float32),
        pltpu.VMEM((SC_CH, F), jnp.float32),
        pltpu.VMEM((NPW + 1, F), jnp.float32),
        pltpu.SemaphoreType.DMA,
        pltpu.SemaphoreType.DMA,
    ],
)
def _sc_segmax(y_hbm, dst_hbm, bounds_hbm, x_hbm,
               bnd, dbuf0, dbuf1, ybuf0, ybuf1, acc, s0, s1):
    w = _wid()
    pltpu.sync_copy(bounds_hbm, bnd)
    nbase = w * NPW
    zero = jnp.zeros((16,), jnp.float32)

    def zrow(r, _):
        for j in range(8):
            acc[r, pl.ds(j * 16, 16)] = zero
        return 0

    lax.fori_loop(0, NPW + 1, zrow, 0)

    bv = bnd[pl.ds(w, 16)]
    lo = bv[0]
    hi = bv[1]
    lo8 = (lo // 8) * 8
    nch_raw = (hi - lo8 + SC_CH - 1) // SC_CH
    nch = jnp.maximum(2 * ((nch_raw + 1) // 2), 2)

    dbufs = (dbuf0, dbuf1)
    ybufs = (ybuf0, ybuf1)
    sems = (s0, s1)

    def cbase(k):
        return jnp.minimum(lo8 + k * SC_CH, N_EDGES - SC_CH)

    def issue(k, p):
        base = cbase(k)
        pltpu.async_copy(dst_hbm.at[pl.ds(base, SC_CH)], dbufs[p], sems[p])
        pltpu.async_copy(y_hbm.at[pl.ds(base, SC_CH)], ybufs[p], sems[p])

    issue(0, 0)
    issue(1, 1)

    def flush(cur_r, a):
        for j in range(8):
            sl = pl.ds(j * 16, 16)
            acc[cur_r, sl] = jnp.maximum(acc[cur_r, sl], a[j])

    def pair(i, carry):
        for p in range(2):
            k = 2 * i + p
            pltpu.make_async_copy(dst_hbm.at[pl.ds(0, SC_CH)], dbufs[p], sems[p]).wait()
            pltpu.make_async_copy(y_hbm.at[pl.ds(0, SC_CH)], ybufs[p], sems[p]).wait()

            def grp(g, carry, p=p):
                dvec = dbufs[p][pl.ds(g * 16, 16)]
                for ii in range(16):
                    d = dvec[ii]
                    r = d - nbase
                    r = jnp.where((r >= 0) & (r < NPW), r, NPW)
                    e = g * 16 + ii
                    yv = [ybufs[p][e, pl.ds(j * 16, 16)] for j in range(8)]
                    cur_r = carry[0]
                    a = carry[1:]
                    change = r != cur_r

                    @pl.when(change)
                    def _(cur_r=cur_r, a=a):
                        flush(cur_r, a)

                    carry = (r,) + tuple(
                        jnp.where(change, yv[j], jnp.maximum(a[j], yv[j]))
                        for j in range(8))
                return carry

            carry = lax.fori_loop(0, SC_CH // 16, grp, carry)

            @pl.when(k + 2 < nch)
            def _(k=k, p=p):
                issue(k + 2, p)
        return carry

    init = (jnp.int32(NPW),) + tuple(zero for _ in range(8))
    carry = lax.fori_loop(0, nch // 2, pair, init)
    flush(carry[0], carry[1:])
    pltpu.sync_copy(acc.at[pl.ds(0, NPW)], x_hbm.at[pl.ds(nbase, NPW)])


def kernel(x, edge_index, batch, W1a, b1a, W1b, b1b, W2a, b2a, W2b, b2b,
           W3a, b3a, W3b, b3b, Wc1, bc1, Wc2, bc2, Wc3, bc3):
    src = edge_index[0]
    dst = edge_index[1]
    sdst, ssrc = lax.sort((dst, src), num_keys=1)
    starts = jnp.arange(33, dtype=jnp.int32) * NPW
    bounds = jnp.searchsorted(sdst, starts).astype(jnp.int32)
    bounds = jnp.zeros((40,), jnp.int32).at[:33].set(bounds)

    def layer(xin, Wa, ba, Wb, bb):
        A, B = _node_proj(xin, Wa, ba)
        H = _sc_gather_combine(A, B, ssrc, sdst)
        Y = _edge_mlp(H, Wb, bb)
        xp = _sc_segmax(Y, sdst, bounds)
        return xp[:N_NODES]

    h1 = layer(x, W1a, b1a, W1b, b1b)
    h2 = layer(h1, W2a, b2a, W2b, b2b)
    h3 = layer(h2, W3a, b3a, W3b, b3b)

    ones = jnp.ones((N_NODES,), dtype=jnp.float32)
    counts = jax.ops.segment_sum(ones, batch, num_segments=N_GRAPHS)
    x_sum = jax.ops.segment_sum(h3, batch, num_segments=N_GRAPHS)
    x_mean = x_sum / jnp.clip(counts, 1.0)[:, None]
    x_max = jax.ops.segment_max(h3, batch, num_segments=N_GRAPHS)
    x_max = jnp.where(jnp.isfinite(x_max), x_max, 0.0)
    g = jnp.concatenate([x_mean, x_max], axis=1)
    h = jax.nn.relu(g @ Wc1 + bc1)
    h = jax.nn.relu(h @ Wc2 + bc2)
    out = h @ Wc3 + bc3
    return jnp.squeeze(out, axis=-1)


# speedup vs baseline: 3.0272x; 1.0257x over previous
import functools

import jax
import jax.numpy as jnp
from jax import lax
from jax.experimental import pallas as pl
from jax.experimental.pallas import tpu as pltpu
from jax.experimental.pallas import tpu_sc as plsc

N_NODES = 10000
N_GRAPHS = 64
N_EDGES = 320000
F = 128

NC, NS = 2, 16
NW = NC * NS
NPW = 320
NPAD = NW * NPW
GC = 128
NCHUNKS = N_EDGES // GC
SC_CH = 256

_mesh = plsc.VectorSubcoreMesh(core_axis_name="c", subcore_axis_name="s")


def _wid():
    return lax.axis_index("s") * NC + lax.axis_index("c")


def _proj_body(x_ref, w_ref, ba_ref, a_ref, b_ref):
    ab = jnp.dot(x_ref[...], w_ref[...], preferred_element_type=jnp.float32)
    a_ref[...] = ab[:, :F] + ba_ref[...]
    b_ref[...] = ab[:, F:]


def _node_proj(x, Wa, ba):
    U = Wa[:F] - Wa[F:]
    V = Wa[F:]
    W2 = jnp.concatenate([U, V], axis=1)
    blk = 2000
    return pl.pallas_call(
        _proj_body,
        grid=(N_NODES // blk,),
        in_specs=[
            pl.BlockSpec((blk, F), lambda i: (i, 0)),
            pl.BlockSpec((F, 2 * F), lambda i: (0, 0)),
            pl.BlockSpec((F,), lambda i: (0,)),
        ],
        out_specs=[
            pl.BlockSpec((blk, F), lambda i: (i, 0)),
            pl.BlockSpec((blk, F), lambda i: (i, 0)),
        ],
        out_shape=[
            jax.ShapeDtypeStruct((N_NODES, F), jnp.float32),
            jax.ShapeDtypeStruct((N_NODES, F), jnp.float32),
        ],
    )(x, W2, ba)


def _edge_mlp_body(z_ref, w_ref, b_ref, y_ref):
    h = jnp.maximum(z_ref[...], 0.0)
    y_ref[...] = jnp.dot(h, w_ref[...], preferred_element_type=jnp.float32) + b_ref[...]


def _edge_mlp(z, Wb, bb):
    blk = 2000
    return pl.pallas_call(
        _edge_mlp_body,
        grid=(N_EDGES // blk,),
        in_specs=[
            pl.BlockSpec((blk, F), lambda i: (i, 0)),
            pl.BlockSpec((F, F), lambda i: (0, 0)),
            pl.BlockSpec((F,), lambda i: (0,)),
        ],
        out_specs=pl.BlockSpec((blk, F), lambda i: (i, 0)),
        out_shape=jax.ShapeDtypeStruct((N_EDGES, F), jnp.float32),
    )(z, Wb, bb)


def _pool_body(x_ref, bt_ref, wc1_ref, bc1_ref, wc2_ref, bc2_ref,
               wc3_ref, bc3_ref, out_ref):
    xv = x_ref[...]
    bt = bt_ref[...]
    gid = lax.broadcasted_iota(jnp.int32, (N_NODES, N_GRAPHS), 1)
    oh = (gid == bt).astype(jnp.float32)
    sums = lax.dot_general(oh, xv, (((0,), (0,)), ((), ())),
                           preferred_element_type=jnp.float32)
    counts = jnp.sum(oh, axis=0)
    mean = sums / jnp.clip(counts, 1.0)[:, None]
    rows = [jnp.max(jnp.where(bt == g, xv, 0.0), axis=0)
            for g in range(N_GRAPHS)]
    xmax = jnp.stack(rows, axis=0)
    g = jnp.concatenate([mean, xmax], axis=1)
    h = jnp.maximum(jnp.dot(g, wc1_ref[...],
                            preferred_element_type=jnp.float32) + bc1_ref[...], 0.0)
    h = jnp.maximum(jnp.dot(h, wc2_ref[...],
                            preferred_element_type=jnp.float32) + bc2_ref[...], 0.0)
    out_ref[...] = jnp.dot(h, wc3_ref[...],
                           preferred_element_type=jnp.float32) + bc3_ref[...]


def _pool_classify(x, batch2, Wc1, bc1, Wc2, bc2, Wc3, bc3):
    return pl.pallas_call(
        _pool_body,
        out_shape=jax.ShapeDtypeStruct((N_GRAPHS, 1), jnp.float32),
    )(x, batch2, Wc1, bc1, Wc2, bc2, Wc3, bc3)


@functools.partial(
    pl.kernel,
    out_type=jax.ShapeDtypeStruct((N_EDGES, F), jnp.float32),
    mesh=_mesh,
    scratch_types=[
        pltpu.VMEM((80 * GC,), jnp.int32),
        pltpu.VMEM((80 * GC,), jnp.int32),
        pltpu.VMEM((GC, F), jnp.float32),
        pltpu.VMEM((GC, F), jnp.float32),
        pltpu.VMEM((GC, F), jnp.float32),
        pltpu.VMEM((GC, F), jnp.float32),
        pltpu.SemaphoreType.DMA,
        pltpu.SemaphoreType.DMA,
        pltpu.SemaphoreType.DMA,
        pltpu.SemaphoreType.DMA,
    ],
)
def _sc_gather_combine(a_hbm, b_hbm, src_hbm, dst_hbm, h_hbm,
                       sidx2, didx2, abuf0, abuf1, bbuf0, bbuf1,
                       sa0, sa1, sb0, sb1):
    w = _wid()
    clo = 2 * ((w * (NCHUNKS // 2)) // NW)
    chi = 2 * (((w + 1) * (NCHUNKS // 2)) // NW)
    nch = chi - clo

    pltpu.sync_copy(dst_hbm.at[pl.ds(clo * GC, 80 * GC)], didx2)
    pltpu.sync_copy(src_hbm.at[pl.ds(clo * GC, 80 * GC)], sidx2)

    abufs = (abuf0, abuf1)
    bbufs = (bbuf0, bbuf1)
    sas = (sa0, sa1)
    sbs = (sb0, sb1)

    def issue(kk, p):
        pltpu.async_copy(a_hbm.at[didx2.at[pl.ds(kk * GC, GC)]], abufs[p], sas[p])
        pltpu.async_copy(b_hbm.at[sidx2.at[pl.ds(kk * GC, GC)]], bbufs[p], sbs[p])

    issue(0, 0)
    issue(1, 1)

    def pair(i, _):
        for p in range(2):
            kk = 2 * i + p
            pltpu.make_async_copy(a_hbm.at[pl.ds(0, GC)], abufs[p], sas[p]).wait()
            pltpu.make_async_copy(b_hbm.at[pl.ds(0, GC)], bbufs[p], sbs[p]).wait()

            def row(r, _, p=p):
                for j in range(8):
                    sl = pl.ds(j * 16, 16)
                    abufs[p][r, sl] = abufs[p][r, sl] + bbufs[p][r, sl]
                return 0

            lax.fori_loop(0, GC, row, 0)
            pltpu.sync_copy(abufs[p], h_hbm.at[pl.ds((clo + kk) * GC, GC)])

            @pl.when(kk + 2 < nch)
            def _(kk=kk, p=p):
                issue(kk + 2, p)
        return 0

    lax.fori_loop(0, nch // 2, pair, 0)


@functools.partial(
    pl.kernel,
    out_type=jax.ShapeDtypeStruct((NPAD, F), jnp.float32),
    mesh=_mesh,
    scratch_types=[
        pltpu.VMEM((40,), jnp.int32),
        pltpu.VMEM((SC_CH,), jnp.int32),
        pltpu.VMEM((SC_CH,), jnp.int32),
        pltpu.VMEM((SC_CH, F), jnp.float32),
        pltpu.VMEM((SC_CH, F), jnp.float32),
        pltpu.VMEM((NPW + 1, F), jnp.float32),
        pltpu.SemaphoreType.DMA,
        pltpu.SemaphoreType.DMA,
    ],
)
def _sc_segmax(y_hbm, dst_hbm, bounds_hbm, x_hbm,
               bnd, dbuf0, dbuf1, ybuf0, ybuf1, acc, s0, s1):
    w = _wid()
    pltpu.sync_copy(bounds_hbm, bnd)
    nbase = w * NPW
    zero = jnp.zeros((16,), jnp.float32)

    def zrow(r, _):
        for j in range(8):
            acc[r, pl.ds(j * 16, 16)] = zero
        return 0

    lax.fori_loop(0, NPW + 1, zrow, 0)

    bv = bnd[pl.ds(w, 16)]
    lo = bv[0]
    hi = bv[1]
    lo8 = (lo // 8) * 8
    nch_raw = (hi - lo8 + SC_CH - 1) // SC_CH
    nch = jnp.maximum(2 * ((nch_raw + 1) // 2), 2)

    dbufs = (dbuf0, dbuf1)
    ybufs = (ybuf0, ybuf1)
    sems = (s0, s1)

    def cbase(k):
        return jnp.minimum(lo8 + k * SC_CH, N_EDGES - SC_CH)

    def issue(k, p):
        base = cbase(k)
        pltpu.async_copy(dst_hbm.at[pl.ds(base, SC_CH)], dbufs[p], sems[p])
        pltpu.async_copy(y_hbm.at[pl.ds(base, SC_CH)], ybufs[p], sems[p])

    issue(0, 0)
    issue(1, 1)

    def flush(cur_r, a):
        for j in range(8):
            sl = pl.ds(j * 16, 16)
            acc[cur_r, sl] = jnp.maximum(acc[cur_r, sl], a[j])

    def pair(i, carry):
        for p in range(2):
            k = 2 * i + p
            pltpu.make_async_copy(dst_hbm.at[pl.ds(0, SC_CH)], dbufs[p], sems[p]).wait()
            pltpu.make_async_copy(y_hbm.at[pl.ds(0, SC_CH)], ybufs[p], sems[p]).wait()

            def grp(g, carry, p=p):
                dvec = dbufs[p][pl.ds(g * 16, 16)]
                for ii in range(16):
                    d = dvec[ii]
                    r = d - nbase
                    r = jnp.where((r >= 0) & (r < NPW), r, NPW)
                    e = g * 16 + ii
                    yv = [ybufs[p][e, pl.ds(j * 16, 16)] for j in range(8)]
                    cur_r = carry[0]
                    a = carry[1:]
                    change = r != cur_r

                    @pl.when(change)
                    def _(cur_r=cur_r, a=a):
                        flush(cur_r, a)

                    carry = (r,) + tuple(
                        jnp.where(change, yv[j], jnp.maximum(a[j], yv[j]))
                        for j in range(8))
                return carry

            carry = lax.fori_loop(0, SC_CH // 16, grp, carry)

            @pl.when(k + 2 < nch)
            def _(k=k, p=p):
                issue(k + 2, p)
        return carry

    init = (jnp.int32(NPW),) + tuple(zero for _ in range(8))
    carry = lax.fori_loop(0, nch // 2, pair, init)
    flush(carry[0], carry[1:])
    pltpu.sync_copy(acc.at[pl.ds(0, NPW)], x_hbm.at[pl.ds(nbase, NPW)])


def kernel(x, edge_index, batch, W1a, b1a, W1b, b1b, W2a, b2a, W2b, b2b,
           W3a, b3a, W3b, b3b, Wc1, bc1, Wc2, bc2, Wc3, bc3):
    src = edge_index[0]
    dst = edge_index[1]
    sdst, ssrc = lax.sort((dst, src), num_keys=1)
    starts = jnp.arange(33, dtype=jnp.int32) * NPW
    bounds = jnp.searchsorted(sdst, starts).astype(jnp.int32)
    bounds = jnp.zeros((40,), jnp.int32).at[:33].set(bounds)

    def layer(xin, Wa, ba, Wb, bb):
        A, B = _node_proj(xin, Wa, ba)
        H = _sc_gather_combine(A, B, ssrc, sdst)
        Y = _edge_mlp(H, Wb, bb)
        xp = _sc_segmax(Y, sdst, bounds)
        return xp[:N_NODES]

    h1 = layer(x, W1a, b1a, W1b, b1b)
    h2 = layer(h1, W2a, b2a, W2b, b2b)
    h3 = layer(h2, W3a, b3a, W3b, b3b)

    pooled = _pool_classify(h3, batch.reshape(N_NODES, 1),
                            Wc1, bc1, Wc2, bc2, Wc3, bc3)
    return jnp.squeeze(pooled, axis=-1)


# speedup vs baseline: 3.0638x; 1.0121x over previous
import functools

import jax
import jax.numpy as jnp
from jax import lax
from jax.experimental import pallas as pl
from jax.experimental.pallas import tpu as pltpu
from jax.experimental.pallas import tpu_sc as plsc

N_NODES = 10000
N_GRAPHS = 64
N_EDGES = 320000
F = 128

NC, NS = 2, 16
NW = NC * NS
NPW = 320
NPAD = NW * NPW
GC = 128
NCHUNKS = N_EDGES // GC
SC_CH = 256

_mesh = plsc.VectorSubcoreMesh(core_axis_name="c", subcore_axis_name="s")


def _wid():
    return lax.axis_index("s") * NC + lax.axis_index("c")


def _proj_body(x_ref, w_ref, ba_ref, a_ref, b_ref):
    ab = jnp.dot(x_ref[...], w_ref[...], preferred_element_type=jnp.float32)
    a_ref[...] = ab[:, :F] + ba_ref[...]
    b_ref[...] = ab[:, F:]


def _node_proj(x, Wa, ba):
    U = Wa[:F] - Wa[F:]
    V = Wa[F:]
    W2 = jnp.concatenate([U, V], axis=1)
    blk = 2000
    return pl.pallas_call(
        _proj_body,
        grid=(N_NODES // blk,),
        in_specs=[
            pl.BlockSpec((blk, F), lambda i: (i, 0)),
            pl.BlockSpec((F, 2 * F), lambda i: (0, 0)),
            pl.BlockSpec((F,), lambda i: (0,)),
        ],
        out_specs=[
            pl.BlockSpec((blk, F), lambda i: (i, 0)),
            pl.BlockSpec((blk, F), lambda i: (i, 0)),
        ],
        out_shape=[
            jax.ShapeDtypeStruct((N_NODES, F), jnp.float32),
            jax.ShapeDtypeStruct((N_NODES, F), jnp.float32),
        ],
    )(x, W2, ba)


def _edge_mlp_body(z_ref, w_ref, b_ref, y_ref):
    h = jnp.maximum(z_ref[...], 0.0)
    y_ref[...] = jnp.dot(h, w_ref[...], preferred_element_type=jnp.float32) + b_ref[...]


def _edge_mlp(z, Wb, bb):
    blk = 2000
    return pl.pallas_call(
        _edge_mlp_body,
        grid=(N_EDGES // blk,),
        in_specs=[
            pl.BlockSpec((blk, F), lambda i: (i, 0)),
            pl.BlockSpec((F, F), lambda i: (0, 0)),
            pl.BlockSpec((F,), lambda i: (0,)),
        ],
        out_specs=pl.BlockSpec((blk, F), lambda i: (i, 0)),
        out_shape=jax.ShapeDtypeStruct((N_EDGES, F), jnp.float32),
    )(z, Wb, bb)


def _pool_body(x_ref, bt_ref, wc1_ref, bc1_ref, wc2_ref, bc2_ref,
               wc3_ref, bc3_ref, out_ref):
    xv = x_ref[...]
    bt = bt_ref[...]
    gid = lax.broadcasted_iota(jnp.int32, (N_NODES, N_GRAPHS), 1)
    oh = (gid == bt).astype(jnp.float32)
    sums = lax.dot_general(oh, xv, (((0,), (0,)), ((), ())),
                           preferred_element_type=jnp.float32)
    counts = jnp.sum(oh, axis=0)
    mean = sums / jnp.clip(counts, 1.0)[:, None]
    rows = [jnp.max(jnp.where(bt == g, xv, 0.0), axis=0)
            for g in range(N_GRAPHS)]
    xmax = jnp.stack(rows, axis=0)
    g = jnp.concatenate([mean, xmax], axis=1)
    h = jnp.maximum(jnp.dot(g, wc1_ref[...],
                            preferred_element_type=jnp.float32) + bc1_ref[...], 0.0)
    h = jnp.maximum(jnp.dot(h, wc2_ref[...],
                            preferred_element_type=jnp.float32) + bc2_ref[...], 0.0)
    out_ref[...] = jnp.dot(h, wc3_ref[...],
                           preferred_element_type=jnp.float32) + bc3_ref[...]


def _pool_classify(x, batch2, Wc1, bc1, Wc2, bc2, Wc3, bc3):
    return pl.pallas_call(
        _pool_body,
        out_shape=jax.ShapeDtypeStruct((N_GRAPHS, 1), jnp.float32),
    )(x, batch2, Wc1, bc1, Wc2, bc2, Wc3, bc3)


@functools.partial(
    pl.kernel,
    out_type=jax.ShapeDtypeStruct((N_EDGES, F), jnp.float32),
    mesh=_mesh,
    scratch_types=[
        pltpu.VMEM((80 * GC,), jnp.int32),
        pltpu.VMEM((80 * GC,), jnp.int32),
        pltpu.VMEM((GC, F), jnp.float32),
        pltpu.VMEM((GC, F), jnp.float32),
        pltpu.VMEM((GC, F), jnp.float32),
        pltpu.VMEM((GC, F), jnp.float32),
        pltpu.VMEM((GC, F), jnp.float32),
        pltpu.VMEM((GC, F), jnp.float32),
        pltpu.SemaphoreType.DMA,
        pltpu.SemaphoreType.DMA,
        pltpu.SemaphoreType.DMA,
        pltpu.SemaphoreType.DMA,
        pltpu.SemaphoreType.DMA,
        pltpu.SemaphoreType.DMA,
    ],
)
def _sc_gather_combine(a_hbm, b_hbm, src_hbm, dst_hbm, h_hbm,
                       sidx2, didx2, abuf0, abuf1, bbuf0, bbuf1,
                       obuf0, obuf1, sa0, sa1, sb0, sb1, sw0, sw1):
    w = _wid()
    clo = 2 * ((w * (NCHUNKS // 2)) // NW)
    chi = 2 * (((w + 1) * (NCHUNKS // 2)) // NW)
    nch = chi - clo

    pltpu.sync_copy(dst_hbm.at[pl.ds(clo * GC, 80 * GC)], didx2)
    pltpu.sync_copy(src_hbm.at[pl.ds(clo * GC, 80 * GC)], sidx2)

    abufs = (abuf0, abuf1)
    bbufs = (bbuf0, bbuf1)
    obufs = (obuf0, obuf1)
    sas = (sa0, sa1)
    sbs = (sb0, sb1)
    sws = (sw0, sw1)

    def issue(kk, p):
        pltpu.async_copy(a_hbm.at[didx2.at[pl.ds(kk * GC, GC)]], abufs[p], sas[p])
        pltpu.async_copy(b_hbm.at[sidx2.at[pl.ds(kk * GC, GC)]], bbufs[p], sbs[p])

    def wait_gathers(p):
        pltpu.make_async_copy(a_hbm.at[pl.ds(0, GC)], abufs[p], sas[p]).wait()
        pltpu.make_async_copy(b_hbm.at[pl.ds(0, GC)], bbufs[p], sbs[p]).wait()

    def wait_wb(p):
        pltpu.make_async_copy(obufs[p], h_hbm.at[pl.ds(0, GC)], sws[p]).wait()

    def compute(p):
        def row(r, _, p=p):
            for j in range(8):
                sl = pl.ds(j * 16, 16)
                obufs[p][r, sl] = abufs[p][r, sl] + bbufs[p][r, sl]
            return 0

        lax.fori_loop(0, GC, row, 0)

    def wb(kk, p):
        pltpu.async_copy(obufs[p], h_hbm.at[pl.ds((clo + kk) * GC, GC)], sws[p])

    issue(0, 0)
    issue(1, 1)

    for p in range(2):
        wait_gathers(p)
        compute(p)
        wb(p, p)
        issue(p + 2, p)

    def pair(i, _):
        for p in range(2):
            kk = 2 * i + p
            wait_gathers(p)
            wait_wb(p)
            compute(p)
            wb(kk, p)

            @pl.when(kk + 2 < nch)
            def _(kk=kk, p=p):
                issue(kk + 2, p)
        return 0

    lax.fori_loop(1, nch // 2, pair, 0)
    wait_wb(0)
    wait_wb(1)


@functools.partial(
    pl.kernel,
    out_type=jax.ShapeDtypeStruct((NPAD, F), jnp.float32),
    mesh=_mesh,
    scratch_types=[
        pltpu.VMEM((40,), jnp.int32),
        pltpu.VMEM((SC_CH,), jnp.int32),
        pltpu.VMEM((SC_CH,), jnp.int32),
        pltpu.VMEM((SC_CH, F), jnp.float32),
        pltpu.VMEM((SC_CH, F), jnp.float32),
        pltpu.VMEM((NPW + 1, F), jnp.float32),
        pltpu.SemaphoreType.DMA,
        pltpu.SemaphoreType.DMA,
    ],
)
def _sc_segmax(y_hbm, dst_hbm, bounds_hbm, x_hbm,
               bnd, dbuf0, dbuf1, ybuf0, ybuf1, acc, s0, s1):
    w = _wid()
    pltpu.sync_copy(bounds_hbm, bnd)
    nbase = w * NPW
    zero = jnp.zeros((16,), jnp.float32)

    def zrow(r, _):
        for j in range(8):
            acc[r, pl.ds(j * 16, 16)] = zero
        return 0

    lax.fori_loop(0, NPW + 1, zrow, 0)

    bv = bnd[pl.ds(w, 16)]
    lo = bv[0]
    hi = bv[1]
    lo8 = (lo // 8) * 8
    nch_raw = (hi - lo8 + SC_CH - 1) // SC_CH
    nch = jnp.maximum(2 * ((nch_raw + 1) // 2), 2)

    dbufs = (dbuf0, dbuf1)
    ybufs = (ybuf0, ybuf1)
    sems = (s0, s1)

    def cbase(k):
        return jnp.minimum(lo8 + k * SC_CH, N_EDGES - SC_CH)

    def issue(k, p):
        base = cbase(k)
        pltpu.async_copy(dst_hbm.at[pl.ds(base, SC_CH)], dbufs[p], sems[p])
        pltpu.async_copy(y_hbm.at[pl.ds(base, SC_CH)], ybufs[p], sems[p])

    issue(0, 0)
    issue(1, 1)

    def flush(cur_r, a):
        for j in range(8):
            sl = pl.ds(j * 16, 16)
            acc[cur_r, sl] = jnp.maximum(acc[cur_r, sl], a[j])

    def pair(i, carry):
        for p in range(2):
            k = 2 * i + p
            pltpu.make_async_copy(dst_hbm.at[pl.ds(0, SC_CH)], dbufs[p], sems[p]).wait()
            pltpu.make_async_copy(y_hbm.at[pl.ds(0, SC_CH)], ybufs[p], sems[p]).wait()

            def grp(g, carry, p=p):
                dvec = dbufs[p][pl.ds(g * 16, 16)]
                for ii in range(16):
                    d = dvec[ii]
                    r = d - nbase
                    r = jnp.where((r >= 0) & (r < NPW), r, NPW)
                    e = g * 16 + ii
                    yv = [ybufs[p][e, pl.ds(j * 16, 16)] for j in range(8)]
                    cur_r = carry[0]
                    a = carry[1:]
                    change = r != cur_r

                    @pl.when(change)
                    def _(cur_r=cur_r, a=a):
                        flush(cur_r, a)

                    carry = (r,) + tuple(
                        jnp.where(change, yv[j], jnp.maximum(a[j], yv[j]))
                        for j in range(8))
                return carry

            carry = lax.fori_loop(0, SC_CH // 16, grp, carry)

            @pl.when(k + 2 < nch)
            def _(k=k, p=p):
                issue(k + 2, p)
        return carry

    init = (jnp.int32(NPW),) + tuple(zero for _ in range(8))
    carry = lax.fori_loop(0, nch // 2, pair, init)
    flush(carry[0], carry[1:])
    pltpu.sync_copy(acc.at[pl.ds(0, NPW)], x_hbm.at[pl.ds(nbase, NPW)])


def kernel(x, edge_index, batch, W1a, b1a, W1b, b1b, W2a, b2a, W2b, b2b,
           W3a, b3a, W3b, b3b, Wc1, bc1, Wc2, bc2, Wc3, bc3):
    src = edge_index[0]
    dst = edge_index[1]
    sdst, ssrc = lax.sort((dst, src), num_keys=1)
    starts = jnp.arange(33, dtype=jnp.int32) * NPW
    bounds = jnp.searchsorted(sdst, starts).astype(jnp.int32)
    bounds = jnp.zeros((40,), jnp.int32).at[:33].set(bounds)

    def layer(xin, Wa, ba, Wb, bb):
        A, B = _node_proj(xin, Wa, ba)
        H = _sc_gather_combine(A, B, ssrc, sdst)
        Y = _edge_mlp(H, Wb, bb)
        xp = _sc_segmax(Y, sdst, bounds)
        return xp[:N_NODES]

    h1 = layer(x, W1a, b1a, W1b, b1b)
    h2 = layer(h1, W2a, b2a, W2b, b2b)
    h3 = layer(h2, W3a, b3a, W3b, b3b)

    pooled = _pool_classify(h3, batch.reshape(N_NODES, 1),
                            Wc1, bc1, Wc2, bc2, Wc3, bc3)
    return jnp.squeeze(pooled, axis=-1)
